# baseline (speedup 1.0000x reference)
import functools

import jax
import jax.numpy as jnp
from jax import lax
from jax.experimental import pallas as pl
from jax.experimental.pallas import tpu as pltpu
from jax.experimental.pallas import tpu_sc as plsc

N = 10000
D = 256
E = 160000
L = 4
HALF = D // 2

NC = 2
NS = 16

CHUNK = 64
EPT = E // NS
NCHUNK = 159
EPT_PAD = NCHUNK * CHUNK
ACC_ROWS = 10112
ZROWS = ACC_ROWS // NS
OROWS = 632
OLAST = N - 15 * OROWS

BT = 400
GRID = N // BT


def _copy_out(acc, out_hbm, s):
    @pl.when(s < NS - 1)
    def _():
        pltpu.sync_copy(acc.at[pl.ds(s * OROWS, OROWS)],
                        out_hbm.at[pl.ds(s * OROWS, OROWS)])

    @pl.when(s == NS - 1)
    def _():
        pltpu.sync_copy(acc.at[pl.ds((NS - 1) * OROWS, OLAST)],
                        out_hbm.at[pl.ds((NS - 1) * OROWS, OLAST)])


@functools.cache
def _segsum_kernel():
    mesh = plsc.VectorSubcoreMesh(core_axis_name="c", subcore_axis_name="s",
                                  num_cores=NC, num_subcores=NS)
    return functools.partial(
        pl.kernel,
        out_type=(jax.ShapeDtypeStruct((N, HALF), jnp.float32),
                  jax.ShapeDtypeStruct((N, HALF), jnp.float32)),
        mesh=mesh,
        scratch_types=[
            pltpu.VMEM((EPT_PAD,), jnp.int32),
            pltpu.VMEM((EPT_PAD,), jnp.int32),
            pltpu.VMEM((CHUNK, HALF), jnp.float32),
            pltpu.VMEM((CHUNK, HALF), jnp.float32),
            pltpu.VMEM((CHUNK, HALF), jnp.float32),
            pltpu.VMEM_SHARED((ACC_ROWS, HALF), jnp.float32),
            pltpu.SemaphoreType.DMA,
            pltpu.SemaphoreType.DMA,
            pltpu.SemaphoreType.DMA,
            pltpu.SemaphoreType.DMA,
        ],
    )(_segsum_body)


def _segsum_body(src_hbm, dst_hbm, z_hbm, ya_hbm, yb_hbm, s0_hbm, s1_hbm,
                 srcv, dstv, rows0, rows1, rows2, acc, gsem0, gsem1, gsem2,
                 ssem):
    c = lax.axis_index("c")
    s = lax.axis_index("s")

    pltpu.sync_copy(z_hbm, acc.at[pl.ds(s * ZROWS, ZROWS)])
    pltpu.sync_copy(src_hbm.at[s], srcv)
    pltpu.sync_copy(dst_hbm.at[s], dstv)
    plsc.subcore_barrier()

    def run(y_hbm):
        bufs = (rows0, rows1, rows2)
        gsems = (gsem0, gsem1, gsem2)

        def sidx(j):
            return srcv.at[pl.ds(pl.multiple_of(j * CHUNK, 8), CHUNK)]

        def didx(j):
            return dstv.at[pl.ds(pl.multiple_of(j * CHUNK, 8), CHUNK)]

        def g(j, b):
            return pltpu.make_async_copy(y_hbm.at[sidx(j)], bufs[b], gsems[b])

        def sc(j, b):
            return pltpu.make_async_copy(bufs[b], acc.at[didx(j)], ssem)

        g(0, 0).start()
        g(1, 1).start()

        def body(k, _):
            j0 = 3 * k
            g(j0, 0).wait()

            @pl.when(k >= 1)
            def _():
                sc(j0 - 1, 2).wait()

            g(j0 + 2, 2).start()
            pltpu.async_copy(bufs[0], acc.at[didx(j0)], ssem, add=True)
            g(j0 + 1, 1).wait()
            sc(j0, 0).wait()

            @pl.when(j0 + 3 < NCHUNK)
            def _():
                g(j0 + 3, 0).start()

            pltpu.async_copy(bufs[1], acc.at[didx(j0 + 1)], ssem, add=True)
            g(j0 + 2, 2).wait()
            sc(j0 + 1, 1).wait()

            @pl.when(j0 + 4 < NCHUNK)
            def _():
                g(j0 + 4, 1).start()

            pltpu.async_copy(bufs[2], acc.at[didx(j0 + 2)], ssem, add=True)
            return ()

        lax.fori_loop(0, NCHUNK // 3, body, (), unroll=False)
        sc(NCHUNK - 1, 2).wait()

    @pl.when(c == 0)
    def _():
        run(ya_hbm)

    @pl.when(c == 1)
    def _():
        run(yb_hbm)

    plsc.subcore_barrier()

    @pl.when(c == 0)
    def _():
        _copy_out(acc, s0_hbm, s)

    @pl.when(c == 1)
    def _():
        _copy_out(acc, s1_hbm, s)


DEG_RING = 4


@functools.cache
def _deg_kernel():
    mesh = plsc.VectorSubcoreMesh(core_axis_name="c", subcore_axis_name="s",
                                  num_cores=NC, num_subcores=NS)
    return functools.partial(
        pl.kernel,
        out_type=(jax.ShapeDtypeStruct((N, HALF), jnp.float32),
                  jax.ShapeDtypeStruct((N, HALF), jnp.float32)),
        mesh=mesh,
        scratch_types=[
            pltpu.VMEM((NCHUNK, CHUNK), jnp.int32),
            pltpu.VMEM((CHUNK, HALF), jnp.float32),
            pltpu.VMEM_SHARED((ACC_ROWS, HALF), jnp.float32),
            pltpu.SemaphoreType.DMA,
        ],
    )(_deg_body)


def _deg_body(dst_hbm, ones_hbm, z_hbm, da_hbm, db_hbm, dstv, onesv, acc, ssem):
    c = lax.axis_index("c")
    s = lax.axis_index("s")

    pltpu.sync_copy(z_hbm, acc.at[pl.ds(s * ZROWS, ZROWS)])
    pltpu.sync_copy(ones_hbm, onesv)
    pltpu.sync_copy(dst_hbm.at[s], dstv)
    plsc.subcore_barrier()

    lo = c * (NCHUNK // 2)
    hi = lax.select(c == 0, NCHUNK // 2, NCHUNK)

    def scatter(j):
        return pltpu.make_async_copy(onesv, acc.at[dstv.at[j]], ssem)

    def body(j, _):
        @pl.when(j - DEG_RING >= lo)
        def _():
            scatter(j - DEG_RING).wait()

        pltpu.async_copy(onesv, acc.at[dstv.at[j]], ssem, add=True)
        return ()

    lax.fori_loop(lo, hi, body, (), unroll=False)

    def drain(j, _):
        @pl.when(j >= lo)
        def _():
            scatter(j).wait()
        return ()

    lax.fori_loop(hi - DEG_RING, hi, drain, (), unroll=False)
    plsc.subcore_barrier()

    @pl.when(c == 0)
    def _():
        _copy_out(acc, da_hbm, s)

    @pl.when(c == 1)
    def _():
        _copy_out(acc, db_hbm, s)


def _silu(v):
    return v * jax.nn.sigmoid(v)


def _dinv_of(da_blk, db_blk):
    return lax.rsqrt(da_blk[:, 0:1] + db_blk[:, 0:1] + 1.0)


def _tc_in_kernel(x_r, da_r, db_r, wi_r, bi_r, wg_r, h_r, ya_r, yb_r):
    dinv = _dinv_of(da_r, db_r)
    h = jnp.dot(x_r[...], wi_r[...], preferred_element_type=jnp.float32) + bi_r[...]
    xw = jnp.dot(h, wg_r[...], preferred_element_type=jnp.float32)
    y = xw * dinv
    h_r[...] = h
    ya_r[...] = y[:, :HALF]
    yb_r[...] = y[:, HALF:]


def _post_common(h_r, s0_r, s1_r, ya_r, yb_r, da_r, db_r, bg_r, bng_r, bnb_r,
                 lng_r, lnb_r, w1_r, w2_r):
    dinv = _dinv_of(da_r, db_r)
    S = jnp.concatenate([s0_r[...], s1_r[...]], axis=1)
    y = jnp.concatenate([ya_r[...], yb_r[...]], axis=1)
    agg = dinv * (S + y) + bg_r[...]
    bn = agg * lax.rsqrt(jnp.float32(1.0 + 1e-5)) * bng_r[...] + bnb_r[...]
    hm = _silu(bn) + h_r[...]
    mu = jnp.mean(hm, axis=-1, keepdims=True)
    var = jnp.mean((hm - mu) ** 2, axis=-1, keepdims=True)
    xn = (hm - mu) * lax.rsqrt(var + 1e-5) * lng_r[...] + lnb_r[...]
    u = _silu(jnp.dot(xn, w1_r[...], preferred_element_type=jnp.float32))
    return jnp.dot(u, w2_r[...], preferred_element_type=jnp.float32) + hm


def _tc_mid_kernel(h_r, s0_r, s1_r, ya_r, yb_r, da_r, db_r, bg_r, bng_r, bnb_r,
                   lng_r, lnb_r, w1_r, w2_r, wgn_r,
                   ho_r, yao_r, ybo_r):
    h2 = _post_common(h_r, s0_r, s1_r, ya_r, yb_r, da_r, db_r, bg_r, bng_r,
                      bnb_r, lng_r, lnb_r, w1_r, w2_r)
    dinv = _dinv_of(da_r, db_r)
    y2 = jnp.dot(h2, wgn_r[...], preferred_element_type=jnp.float32) * dinv
    ho_r[...] = h2
    yao_r[...] = y2[:, :HALF]
    ybo_r[...] = y2[:, HALF:]


def _tc_last_kernel(h_r, s0_r, s1_r, ya_r, yb_r, da_r, db_r, bg_r, bng_r,
                    bnb_r, lng_r, lnb_r, w1_r, w2_r, wo_r, bo_r, out_r):
    h2 = _post_common(h_r, s0_r, s1_r, ya_r, yb_r, da_r, db_r, bg_r, bng_r,
                      bnb_r, lng_r, lnb_r, w1_r, w2_r)
    out_r[...] = jnp.dot(h2, wo_r[...], preferred_element_type=jnp.float32) + bo_r[...]


def _row_spec(w):
    return pl.BlockSpec((BT, w), lambda i: (i, 0))


def _full_spec(r, w):
    return pl.BlockSpec((r, w), lambda i: (0, 0))


def _sds(r, w):
    return jax.ShapeDtypeStruct((r, w), jnp.float32)


def kernel(x, edge_index, W_in, b_in, W_gcn, b_gcn, bn_gamma, bn_beta,
           ln_gamma, ln_beta, W1, W2, W_out, b_out):
    src = edge_index[0].reshape(NS, EPT)
    dst = edge_index[1].reshape(NS, EPT)
    src_pad = jnp.pad(src, ((0, 0), (0, EPT_PAD - EPT)))
    dst_pad = jnp.pad(dst, ((0, 0), (0, EPT_PAD - EPT)), constant_values=N)
    dst_pad3 = dst_pad.reshape(NS, NCHUNK, CHUNK)
    z_half = jnp.zeros((ZROWS, HALF), jnp.float32)
    ones_chunk = jnp.ones((CHUNK, HALF), jnp.float32)

    da, db = _deg_kernel()(dst_pad3, ones_chunk, z_half)

    b_in2 = b_in.reshape(1, D)
    b_out2 = b_out.reshape(1, D)

    h, ya, yb = pl.pallas_call(
        _tc_in_kernel,
        grid=(GRID,),
        in_specs=[_row_spec(D), _row_spec(HALF), _row_spec(HALF),
                  _full_spec(D, D), _full_spec(1, D), _full_spec(D, D)],
        out_specs=[_row_spec(D), _row_spec(HALF), _row_spec(HALF)],
        out_shape=[_sds(N, D), _sds(N, HALF), _sds(N, HALF)],
    )(x, da, db, W_in, b_in2, W_gcn[0])

    mid = pl.pallas_call(
        _tc_mid_kernel,
        grid=(GRID,),
        in_specs=[_row_spec(D)] + [_row_spec(HALF)] * 6
                 + [_full_spec(1, D)] * 5
                 + [_full_spec(D, D)] * 3,
        out_specs=[_row_spec(D), _row_spec(HALF), _row_spec(HALF)],
        out_shape=[_sds(N, D), _sds(N, HALF), _sds(N, HALF)],
    )

    last = pl.pallas_call(
        _tc_last_kernel,
        grid=(GRID,),
        in_specs=[_row_spec(D)] + [_row_spec(HALF)] * 6
                 + [_full_spec(1, D)] * 5
                 + [_full_spec(D, D)] * 2
                 + [_full_spec(D, D), _full_spec(1, D)],
        out_specs=_row_spec(D),
        out_shape=_sds(N, D),
    )

    for i in range(L):
        s0, s1 = _segsum_kernel()(src_pad, dst_pad, z_half, ya, yb)
        norms = (b_gcn[i].reshape(1, D), bn_gamma[i].reshape(1, D),
                 bn_beta[i].reshape(1, D), ln_gamma[i].reshape(1, D),
                 ln_beta[i].reshape(1, D))
        if i < L - 1:
            h, ya, yb = mid(h, s0, s1, ya, yb, da, db, *norms,
                            W1[i], W2[i], W_gcn[i + 1])
        else:
            out = last(h, s0, s1, ya, yb, da, db, *norms,
                       W1[i], W2[i], W_out, b_out2)
    return out


# speedup vs baseline: 1.1306x; 1.1306x over previous
import functools

import jax
import jax.numpy as jnp
from jax import lax
from jax.experimental import pallas as pl
from jax.experimental.pallas import tpu as pltpu
from jax.experimental.pallas import tpu_sc as plsc

N = 10000
D = 256
E = 160000
L = 4
HALF = D // 2

NC = 2
NS = 16

CHUNK = 112
EPT = E // NS
NCHUNK = 90
EPT_PAD = NCHUNK * CHUNK
ACC_ROWS = 10112
ZROWS = ACC_ROWS // NS
OROWS = 632
OLAST = N - 15 * OROWS

BT = 400
GRID = N // BT


def _copy_out(acc, out_hbm, s):
    @pl.when(s < NS - 1)
    def _():
        pltpu.sync_copy(acc.at[pl.ds(s * OROWS, OROWS)],
                        out_hbm.at[pl.ds(s * OROWS, OROWS)])

    @pl.when(s == NS - 1)
    def _():
        pltpu.sync_copy(acc.at[pl.ds((NS - 1) * OROWS, OLAST)],
                        out_hbm.at[pl.ds((NS - 1) * OROWS, OLAST)])


@functools.cache
def _segsum_kernel():
    mesh = plsc.VectorSubcoreMesh(core_axis_name="c", subcore_axis_name="s",
                                  num_cores=NC, num_subcores=NS)
    return functools.partial(
        pl.kernel,
        out_type=(jax.ShapeDtypeStruct((N, HALF), jnp.float32),
                  jax.ShapeDtypeStruct((N, HALF), jnp.float32)),
        mesh=mesh,
        scratch_types=[
            pltpu.VMEM((EPT_PAD,), jnp.int32),
            pltpu.VMEM((EPT_PAD,), jnp.int32),
            pltpu.VMEM((CHUNK, HALF), jnp.float32),
            pltpu.VMEM((CHUNK, HALF), jnp.float32),
            pltpu.VMEM_SHARED((ACC_ROWS, HALF), jnp.float32),
            pltpu.SemaphoreType.DMA,
            pltpu.SemaphoreType.DMA,
        ],
    )(_segsum_body)


def _segsum_body(src_hbm, dst_hbm, z_hbm, ya_hbm, yb_hbm, s0_hbm, s1_hbm,
                 srcv, dstv, rows0, rows1, acc, gsem, ssem):
    c = lax.axis_index("c")
    s = lax.axis_index("s")

    pltpu.sync_copy(z_hbm, acc.at[pl.ds(s * ZROWS, ZROWS)])
    pltpu.sync_copy(src_hbm.at[s], srcv)
    pltpu.sync_copy(dst_hbm.at[s], dstv)
    plsc.subcore_barrier()

    def run(y_hbm):
        def sidx(j):
            return srcv.at[pl.ds(pl.multiple_of(j * CHUNK, 8), CHUNK)]

        def didx(j):
            return dstv.at[pl.ds(pl.multiple_of(j * CHUNK, 8), CHUNK)]

        def gather(j, buf):
            return pltpu.make_async_copy(y_hbm.at[sidx(j)], buf, gsem)

        def scatter(j, buf):
            return pltpu.make_async_copy(buf, acc.at[didx(j)], ssem)

        gather(0, rows0).start()

        def body(k, _):
            j0 = 2 * k
            gather(j0, rows0).wait()

            @pl.when(k >= 1)
            def _():
                scatter(j0 - 1, rows1).wait()

            gather(j0 + 1, rows1).start()
            pltpu.async_copy(rows0, acc.at[didx(j0)], ssem, add=True)
            gather(j0 + 1, rows1).wait()
            scatter(j0, rows0).wait()

            @pl.when(j0 + 2 < NCHUNK)
            def _():
                gather(j0 + 2, rows0).start()

            pltpu.async_copy(rows1, acc.at[didx(j0 + 1)], ssem, add=True)
            return ()

        lax.fori_loop(0, NCHUNK // 2, body, (), unroll=False)
        scatter(NCHUNK - 1, rows1).wait()

    @pl.when(c == 0)
    def _():
        run(ya_hbm)

    @pl.when(c == 1)
    def _():
        run(yb_hbm)

    plsc.subcore_barrier()

    @pl.when(c == 0)
    def _():
        _copy_out(acc, s0_hbm, s)

    @pl.when(c == 1)
    def _():
        _copy_out(acc, s1_hbm, s)


DEG_RING = 4


@functools.cache
def _deg_kernel():
    mesh = plsc.VectorSubcoreMesh(core_axis_name="c", subcore_axis_name="s",
                                  num_cores=NC, num_subcores=NS)
    return functools.partial(
        pl.kernel,
        out_type=(jax.ShapeDtypeStruct((N, HALF), jnp.float32),
                  jax.ShapeDtypeStruct((N, HALF), jnp.float32)),
        mesh=mesh,
        scratch_types=[
            pltpu.VMEM((NCHUNK, CHUNK), jnp.int32),
            pltpu.VMEM((CHUNK, HALF), jnp.float32),
            pltpu.VMEM_SHARED((ACC_ROWS, HALF), jnp.float32),
            pltpu.SemaphoreType.DMA,
        ],
    )(_deg_body)


def _deg_body(dst_hbm, ones_hbm, z_hbm, da_hbm, db_hbm, dstv, onesv, acc, ssem):
    c = lax.axis_index("c")
    s = lax.axis_index("s")

    pltpu.sync_copy(z_hbm, acc.at[pl.ds(s * ZROWS, ZROWS)])
    pltpu.sync_copy(ones_hbm, onesv)
    pltpu.sync_copy(dst_hbm.at[s], dstv)
    plsc.subcore_barrier()

    lo = c * (NCHUNK // 2)
    hi = lax.select(c == 0, NCHUNK // 2, NCHUNK)

    def scatter(j):
        return pltpu.make_async_copy(onesv, acc.at[dstv.at[j]], ssem)

    def body(j, _):
        @pl.when(j - DEG_RING >= lo)
        def _():
            scatter(j - DEG_RING).wait()

        pltpu.async_copy(onesv, acc.at[dstv.at[j]], ssem, add=True)
        return ()

    lax.fori_loop(lo, hi, body, (), unroll=False)

    def drain(j, _):
        @pl.when(j >= lo)
        def _():
            scatter(j).wait()
        return ()

    lax.fori_loop(hi - DEG_RING, hi, drain, (), unroll=False)
    plsc.subcore_barrier()

    @pl.when(c == 0)
    def _():
        _copy_out(acc, da_hbm, s)

    @pl.when(c == 1)
    def _():
        _copy_out(acc, db_hbm, s)


def _silu(v):
    return v * jax.nn.sigmoid(v)


def _dinv_of(da_blk, db_blk):
    return lax.rsqrt(da_blk[:, 0:1] + db_blk[:, 0:1] + 1.0)


def _tc_in_kernel(x_r, da_r, db_r, wi_r, bi_r, wg_r, h_r, ya_r, yb_r):
    dinv = _dinv_of(da_r, db_r)
    h = jnp.dot(x_r[...], wi_r[...], preferred_element_type=jnp.float32) + bi_r[...]
    xw = jnp.dot(h, wg_r[...], preferred_element_type=jnp.float32)
    y = xw * dinv
    h_r[...] = h
    ya_r[...] = y[:, :HALF]
    yb_r[...] = y[:, HALF:]


def _post_common(h_r, s0_r, s1_r, ya_r, yb_r, da_r, db_r, bg_r, bng_r, bnb_r,
                 lng_r, lnb_r, w1_r, w2_r):
    dinv = _dinv_of(da_r, db_r)
    S = jnp.concatenate([s0_r[...], s1_r[...]], axis=1)
    y = jnp.concatenate([ya_r[...], yb_r[...]], axis=1)
    agg = dinv * (S + y) + bg_r[...]
    bn = agg * lax.rsqrt(jnp.float32(1.0 + 1e-5)) * bng_r[...] + bnb_r[...]
    hm = _silu(bn) + h_r[...]
    mu = jnp.mean(hm, axis=-1, keepdims=True)
    var = jnp.mean((hm - mu) ** 2, axis=-1, keepdims=True)
    xn = (hm - mu) * lax.rsqrt(var + 1e-5) * lng_r[...] + lnb_r[...]
    u = _silu(jnp.dot(xn, w1_r[...], preferred_element_type=jnp.float32))
    return jnp.dot(u, w2_r[...], preferred_element_type=jnp.float32) + hm


def _tc_mid_kernel(h_r, s0_r, s1_r, ya_r, yb_r, da_r, db_r, bg_r, bng_r, bnb_r,
                   lng_r, lnb_r, w1_r, w2_r, wgn_r,
                   ho_r, yao_r, ybo_r):
    h2 = _post_common(h_r, s0_r, s1_r, ya_r, yb_r, da_r, db_r, bg_r, bng_r,
                      bnb_r, lng_r, lnb_r, w1_r, w2_r)
    dinv = _dinv_of(da_r, db_r)
    y2 = jnp.dot(h2, wgn_r[...], preferred_element_type=jnp.float32) * dinv
    ho_r[...] = h2
    yao_r[...] = y2[:, :HALF]
    ybo_r[...] = y2[:, HALF:]


def _tc_last_kernel(h_r, s0_r, s1_r, ya_r, yb_r, da_r, db_r, bg_r, bng_r,
                    bnb_r, lng_r, lnb_r, w1_r, w2_r, wo_r, bo_r, out_r):
    h2 = _post_common(h_r, s0_r, s1_r, ya_r, yb_r, da_r, db_r, bg_r, bng_r,
                      bnb_r, lng_r, lnb_r, w1_r, w2_r)
    out_r[...] = jnp.dot(h2, wo_r[...], preferred_element_type=jnp.float32) + bo_r[...]


def _row_spec(w):
    return pl.BlockSpec((BT, w), lambda i: (i, 0))


def _full_spec(r, w):
    return pl.BlockSpec((r, w), lambda i: (0, 0))


def _sds(r, w):
    return jax.ShapeDtypeStruct((r, w), jnp.float32)


def kernel(x, edge_index, W_in, b_in, W_gcn, b_gcn, bn_gamma, bn_beta,
           ln_gamma, ln_beta, W1, W2, W_out, b_out):
    src = edge_index[0].reshape(NS, EPT)
    dst = edge_index[1].reshape(NS, EPT)
    src_pad = jnp.pad(src, ((0, 0), (0, EPT_PAD - EPT)))
    dst_pad = jnp.pad(dst, ((0, 0), (0, EPT_PAD - EPT)), constant_values=N)
    dst_pad3 = dst_pad.reshape(NS, NCHUNK, CHUNK)
    z_half = jnp.zeros((ZROWS, HALF), jnp.float32)
    ones_chunk = jnp.ones((CHUNK, HALF), jnp.float32)

    da, db = _deg_kernel()(dst_pad3, ones_chunk, z_half)

    b_in2 = b_in.reshape(1, D)
    b_out2 = b_out.reshape(1, D)

    h, ya, yb = pl.pallas_call(
        _tc_in_kernel,
        grid=(GRID,),
        in_specs=[_row_spec(D), _row_spec(HALF), _row_spec(HALF),
                  _full_spec(D, D), _full_spec(1, D), _full_spec(D, D)],
        out_specs=[_row_spec(D), _row_spec(HALF), _row_spec(HALF)],
        out_shape=[_sds(N, D), _sds(N, HALF), _sds(N, HALF)],
    )(x, da, db, W_in, b_in2, W_gcn[0])

    mid = pl.pallas_call(
        _tc_mid_kernel,
        grid=(GRID,),
        in_specs=[_row_spec(D)] + [_row_spec(HALF)] * 6
                 + [_full_spec(1, D)] * 5
                 + [_full_spec(D, D)] * 3,
        out_specs=[_row_spec(D), _row_spec(HALF), _row_spec(HALF)],
        out_shape=[_sds(N, D), _sds(N, HALF), _sds(N, HALF)],
    )

    last = pl.pallas_call(
        _tc_last_kernel,
        grid=(GRID,),
        in_specs=[_row_spec(D)] + [_row_spec(HALF)] * 6
                 + [_full_spec(1, D)] * 5
                 + [_full_spec(D, D)] * 2
                 + [_full_spec(D, D), _full_spec(1, D)],
        out_specs=_row_spec(D),
        out_shape=_sds(N, D),
    )

    for i in range(L):
        s0, s1 = _segsum_kernel()(src_pad, dst_pad, z_half, ya, yb)
        norms = (b_gcn[i].reshape(1, D), bn_gamma[i].reshape(1, D),
                 bn_beta[i].reshape(1, D), ln_gamma[i].reshape(1, D),
                 ln_beta[i].reshape(1, D))
        if i < L - 1:
            h, ya, yb = mid(h, s0, s1, ya, yb, da, db, *norms,
                            W1[i], W2[i], W_gcn[i + 1])
        else:
            out = last(h, s0, s1, ya, yb, da, db, *norms,
                       W1[i], W2[i], W_out, b_out2)
    return out


# speedup vs baseline: 1.3357x; 1.1814x over previous
import functools

import jax
import jax.numpy as jnp
from jax import lax
from jax.experimental import pallas as pl
from jax.experimental.pallas import tpu as pltpu
from jax.experimental.pallas import tpu_sc as plsc

N = 10000
D = 256
E = 160000
L = 4
HALF = D // 2

NC = 2
NS = 16

CHUNK = 112
EPT = E // NS
NCHUNK = 90
EPT_PAD = NCHUNK * CHUNK
GRPC = 3 * CHUNK
NGROUP = NCHUNK // 3
ACC_ROWS = 10112
ZROWS = ACC_ROWS // NS
OROWS = 632
OLAST = N - 15 * OROWS

BT = 400
GRID = N // BT


def _copy_out(acc, out_hbm, s):
    @pl.when(s < NS - 1)
    def _():
        pltpu.sync_copy(acc.at[pl.ds(s * OROWS, OROWS)],
                        out_hbm.at[pl.ds(s * OROWS, OROWS)])

    @pl.when(s == NS - 1)
    def _():
        pltpu.sync_copy(acc.at[pl.ds((NS - 1) * OROWS, OLAST)],
                        out_hbm.at[pl.ds((NS - 1) * OROWS, OLAST)])


@functools.cache
def _segsum_kernel():
    mesh = plsc.VectorSubcoreMesh(core_axis_name="c", subcore_axis_name="s",
                                  num_cores=NC, num_subcores=NS)
    return functools.partial(
        pl.kernel,
        out_type=(jax.ShapeDtypeStruct((N, HALF), jnp.float32),
                  jax.ShapeDtypeStruct((N, HALF), jnp.float32)),
        mesh=mesh,
        scratch_types=[
            pltpu.VMEM((2 * GRPC,), jnp.int32),
            pltpu.VMEM((2 * GRPC,), jnp.int32),
            pltpu.VMEM((CHUNK, HALF), jnp.float32),
            pltpu.VMEM((CHUNK, HALF), jnp.float32),
            pltpu.VMEM((CHUNK, HALF), jnp.float32),
            pltpu.VMEM_SHARED((ACC_ROWS, HALF), jnp.float32),
            pltpu.SemaphoreType.DMA,
            pltpu.SemaphoreType.DMA,
            pltpu.SemaphoreType.DMA,
            pltpu.SemaphoreType.DMA,
            pltpu.SemaphoreType.DMA,
        ],
    )(_segsum_body)


def _segsum_body(src_hbm, dst_hbm, z_hbm, ya_hbm, yb_hbm, s0_hbm, s1_hbm,
                 srcr, dstr, rows0, rows1, rows2, acc, gsem0, gsem1, gsem2,
                 ssem, isem):
    c = lax.axis_index("c")
    s = lax.axis_index("s")
    base = s * EPT_PAD

    pltpu.sync_copy(z_hbm, acc.at[pl.ds(s * ZROWS, ZROWS)])
    pltpu.sync_copy(src_hbm.at[pl.ds(base, GRPC)], srcr.at[pl.ds(0, GRPC)])
    pltpu.sync_copy(dst_hbm.at[pl.ds(base, GRPC)], dstr.at[pl.ds(0, GRPC)])
    plsc.subcore_barrier()

    def run(y_hbm):
        bufs = (rows0, rows1, rows2)
        gsems = (gsem0, gsem1, gsem2)

        def roff(k, ch):
            return pl.multiple_of(lax.rem(k, 2) * GRPC + ch * CHUNK, 8)

        def g(k, ch, b):
            idx = srcr.at[pl.ds(roff(k, ch), CHUNK)]
            return pltpu.make_async_copy(y_hbm.at[idx], bufs[b], gsems[b])

        def didx(k, ch):
            return dstr.at[pl.ds(roff(k, ch), CHUNK)]

        def sc(k, ch, b):
            return pltpu.make_async_copy(bufs[b], acc.at[didx(k, ch)], ssem)

        def refill(gn, start):
            hb = pl.multiple_of(base + gn * GRPC, 8)
            ho = pl.multiple_of(lax.rem(gn, 2) * GRPC, 8)
            for hbm, ring in ((src_hbm, srcr), (dst_hbm, dstr)):
                d = pltpu.make_async_copy(hbm.at[pl.ds(hb, GRPC)],
                                          ring.at[pl.ds(ho, GRPC)], isem)
                d.start() if start else d.wait()

        g(0, 0, 0).start()
        g(0, 1, 1).start()

        def body(k, _):
            g(k, 0, 0).wait()

            @pl.when(k >= 1)
            def _():
                sc(k - 1, 2, 2).wait()

            @pl.when(k + 1 < NGROUP)
            def _():
                refill(k + 1, True)

            g(k, 2, 2).start()
            pltpu.async_copy(bufs[0], acc.at[didx(k, 0)], ssem, add=True)
            g(k, 1, 1).wait()
            sc(k, 0, 0).wait()

            @pl.when(k + 1 < NGROUP)
            def _():
                refill(k + 1, False)
                g(k + 1, 0, 0).start()

            pltpu.async_copy(bufs[1], acc.at[didx(k, 1)], ssem, add=True)
            g(k, 2, 2).wait()
            sc(k, 1, 1).wait()

            @pl.when(k + 1 < NGROUP)
            def _():
                g(k + 1, 1, 1).start()

            pltpu.async_copy(bufs[2], acc.at[didx(k, 2)], ssem, add=True)
            return ()

        lax.fori_loop(0, NGROUP, body, (), unroll=False)
        sc(NGROUP - 1, 2, 2).wait()

    @pl.when(c == 0)
    def _():
        run(ya_hbm)

    @pl.when(c == 1)
    def _():
        run(yb_hbm)

    plsc.subcore_barrier()

    @pl.when(c == 0)
    def _():
        _copy_out(acc, s0_hbm, s)

    @pl.when(c == 1)
    def _():
        _copy_out(acc, s1_hbm, s)


DEG_RING = 4


@functools.cache
def _deg_kernel():
    mesh = plsc.VectorSubcoreMesh(core_axis_name="c", subcore_axis_name="s",
                                  num_cores=NC, num_subcores=NS)
    return functools.partial(
        pl.kernel,
        out_type=(jax.ShapeDtypeStruct((N, HALF), jnp.float32),
                  jax.ShapeDtypeStruct((N, HALF), jnp.float32)),
        mesh=mesh,
        scratch_types=[
            pltpu.VMEM((NCHUNK, CHUNK), jnp.int32),
            pltpu.VMEM((CHUNK, HALF), jnp.float32),
            pltpu.VMEM_SHARED((ACC_ROWS, HALF), jnp.float32),
            pltpu.SemaphoreType.DMA,
        ],
    )(_deg_body)


def _deg_body(dst_hbm, ones_hbm, z_hbm, da_hbm, db_hbm, dstv, onesv, acc, ssem):
    c = lax.axis_index("c")
    s = lax.axis_index("s")

    pltpu.sync_copy(z_hbm, acc.at[pl.ds(s * ZROWS, ZROWS)])
    pltpu.sync_copy(ones_hbm, onesv)
    pltpu.sync_copy(dst_hbm.at[s], dstv)
    plsc.subcore_barrier()

    lo = c * (NCHUNK // 2)
    hi = lax.select(c == 0, NCHUNK // 2, NCHUNK)

    def scatter(j):
        return pltpu.make_async_copy(onesv, acc.at[dstv.at[j]], ssem)

    def body(j, _):
        @pl.when(j - DEG_RING >= lo)
        def _():
            scatter(j - DEG_RING).wait()

        pltpu.async_copy(onesv, acc.at[dstv.at[j]], ssem, add=True)
        return ()

    lax.fori_loop(lo, hi, body, (), unroll=False)

    def drain(j, _):
        @pl.when(j >= lo)
        def _():
            scatter(j).wait()
        return ()

    lax.fori_loop(hi - DEG_RING, hi, drain, (), unroll=False)
    plsc.subcore_barrier()

    @pl.when(c == 0)
    def _():
        _copy_out(acc, da_hbm, s)

    @pl.when(c == 1)
    def _():
        _copy_out(acc, db_hbm, s)


def _silu(v):
    return v * jax.nn.sigmoid(v)


def _dinv_of(da_blk, db_blk):
    return lax.rsqrt(da_blk[:, 0:1] + db_blk[:, 0:1] + 1.0)


def _tc_in_kernel(x_r, da_r, db_r, wi_r, bi_r, wg_r, h_r, ya_r, yb_r):
    dinv = _dinv_of(da_r, db_r)
    h = jnp.dot(x_r[...], wi_r[...], preferred_element_type=jnp.float32) + bi_r[...]
    xw = jnp.dot(h, wg_r[...], preferred_element_type=jnp.float32)
    y = xw * dinv
    h_r[...] = h
    ya_r[...] = y[:, :HALF]
    yb_r[...] = y[:, HALF:]


def _post_common(h_r, s0_r, s1_r, ya_r, yb_r, da_r, db_r, bg_r, bng_r, bnb_r,
                 lng_r, lnb_r, w1_r, w2_r):
    dinv = _dinv_of(da_r, db_r)
    S = jnp.concatenate([s0_r[...], s1_r[...]], axis=1)
    y = jnp.concatenate([ya_r[...], yb_r[...]], axis=1)
    agg = dinv * (S + y) + bg_r[...]
    bn = agg * lax.rsqrt(jnp.float32(1.0 + 1e-5)) * bng_r[...] + bnb_r[...]
    hm = _silu(bn) + h_r[...]
    mu = jnp.mean(hm, axis=-1, keepdims=True)
    var = jnp.mean((hm - mu) ** 2, axis=-1, keepdims=True)
    xn = (hm - mu) * lax.rsqrt(var + 1e-5) * lng_r[...] + lnb_r[...]
    u = _silu(jnp.dot(xn, w1_r[...], preferred_element_type=jnp.float32))
    return jnp.dot(u, w2_r[...], preferred_element_type=jnp.float32) + hm


def _tc_mid_kernel(h_r, s0_r, s1_r, ya_r, yb_r, da_r, db_r, bg_r, bng_r, bnb_r,
                   lng_r, lnb_r, w1_r, w2_r, wgn_r,
                   ho_r, yao_r, ybo_r):
    h2 = _post_common(h_r, s0_r, s1_r, ya_r, yb_r, da_r, db_r, bg_r, bng_r,
                      bnb_r, lng_r, lnb_r, w1_r, w2_r)
    dinv = _dinv_of(da_r, db_r)
    y2 = jnp.dot(h2, wgn_r[...], preferred_element_type=jnp.float32) * dinv
    ho_r[...] = h2
    yao_r[...] = y2[:, :HALF]
    ybo_r[...] = y2[:, HALF:]


def _tc_last_kernel(h_r, s0_r, s1_r, ya_r, yb_r, da_r, db_r, bg_r, bng_r,
                    bnb_r, lng_r, lnb_r, w1_r, w2_r, wo_r, bo_r, out_r):
    h2 = _post_common(h_r, s0_r, s1_r, ya_r, yb_r, da_r, db_r, bg_r, bng_r,
                      bnb_r, lng_r, lnb_r, w1_r, w2_r)
    out_r[...] = jnp.dot(h2, wo_r[...], preferred_element_type=jnp.float32) + bo_r[...]


def _row_spec(w):
    return pl.BlockSpec((BT, w), lambda i: (i, 0))


def _full_spec(r, w):
    return pl.BlockSpec((r, w), lambda i: (0, 0))


def _sds(r, w):
    return jax.ShapeDtypeStruct((r, w), jnp.float32)


def kernel(x, edge_index, W_in, b_in, W_gcn, b_gcn, bn_gamma, bn_beta,
           ln_gamma, ln_beta, W1, W2, W_out, b_out):
    src = edge_index[0].reshape(NS, EPT)
    dst = edge_index[1].reshape(NS, EPT)
    src_pad = jnp.pad(src, ((0, 0), (0, EPT_PAD - EPT))).reshape(-1)
    dst_pad = jnp.pad(dst, ((0, 0), (0, EPT_PAD - EPT)), constant_values=N)
    dst_pad3 = dst_pad.reshape(NS, NCHUNK, CHUNK)
    dst_pad = dst_pad.reshape(-1)
    z_half = jnp.zeros((ZROWS, HALF), jnp.float32)
    ones_chunk = jnp.ones((CHUNK, HALF), jnp.float32)

    da, db = _deg_kernel()(dst_pad3, ones_chunk, z_half)

    b_in2 = b_in.reshape(1, D)
    b_out2 = b_out.reshape(1, D)

    h, ya, yb = pl.pallas_call(
        _tc_in_kernel,
        grid=(GRID,),
        in_specs=[_row_spec(D), _row_spec(HALF), _row_spec(HALF),
                  _full_spec(D, D), _full_spec(1, D), _full_spec(D, D)],
        out_specs=[_row_spec(D), _row_spec(HALF), _row_spec(HALF)],
        out_shape=[_sds(N, D), _sds(N, HALF), _sds(N, HALF)],
    )(x, da, db, W_in, b_in2, W_gcn[0])

    mid = pl.pallas_call(
        _tc_mid_kernel,
        grid=(GRID,),
        in_specs=[_row_spec(D)] + [_row_spec(HALF)] * 6
                 + [_full_spec(1, D)] * 5
                 + [_full_spec(D, D)] * 3,
        out_specs=[_row_spec(D), _row_spec(HALF), _row_spec(HALF)],
        out_shape=[_sds(N, D), _sds(N, HALF), _sds(N, HALF)],
    )

    last = pl.pallas_call(
        _tc_last_kernel,
        grid=(GRID,),
        in_specs=[_row_spec(D)] + [_row_spec(HALF)] * 6
                 + [_full_spec(1, D)] * 5
                 + [_full_spec(D, D)] * 2
                 + [_full_spec(D, D), _full_spec(1, D)],
        out_specs=_row_spec(D),
        out_shape=_sds(N, D),
    )

    for i in range(L):
        s0, s1 = _segsum_kernel()(src_pad, dst_pad, z_half, ya, yb)
        norms = (b_gcn[i].reshape(1, D), bn_gamma[i].reshape(1, D),
                 bn_beta[i].reshape(1, D), ln_gamma[i].reshape(1, D),
                 ln_beta[i].reshape(1, D))
        if i < L - 1:
            h, ya, yb = mid(h, s0, s1, ya, yb, da, db, *norms,
                            W1[i], W2[i], W_gcn[i + 1])
        else:
            out = last(h, s0, s1, ya, yb, da, db, *norms,
                       W1[i], W2[i], W_out, b_out2)
    return out


# speedup vs baseline: 1.3373x; 1.0012x over previous
import functools

import jax
import jax.numpy as jnp
from jax import lax
from jax.experimental import pallas as pl
from jax.experimental.pallas import tpu as pltpu
from jax.experimental.pallas import tpu_sc as plsc

N = 10000
D = 256
E = 160000
L = 4
HALF = D // 2

NC = 2
NS = 16

CHUNK = 112
EPT = E // NS
NCHUNK = 90
EPT_PAD = NCHUNK * CHUNK
GRPC = 3 * CHUNK
NGROUP = NCHUNK // 3
ACC_ROWS = 10112
ZROWS = ACC_ROWS // NS
OROWS = 632
OLAST = N - 15 * OROWS

BT = 400
GRID = N // BT


def _copy_out(acc, out_hbm, s):
    @pl.when(s < NS - 1)
    def _():
        pltpu.sync_copy(acc.at[pl.ds(s * OROWS, OROWS)],
                        out_hbm.at[pl.ds(s * OROWS, OROWS)])

    @pl.when(s == NS - 1)
    def _():
        pltpu.sync_copy(acc.at[pl.ds((NS - 1) * OROWS, OLAST)],
                        out_hbm.at[pl.ds((NS - 1) * OROWS, OLAST)])


@functools.cache
def _segsum_kernel():
    mesh = plsc.VectorSubcoreMesh(core_axis_name="c", subcore_axis_name="s",
                                  num_cores=NC, num_subcores=NS)
    return functools.partial(
        pl.kernel,
        out_type=(jax.ShapeDtypeStruct((N, HALF), jnp.float32),
                  jax.ShapeDtypeStruct((N, HALF), jnp.float32)),
        mesh=mesh,
        scratch_types=[
            pltpu.VMEM((2 * GRPC,), jnp.int32),
            pltpu.VMEM((2 * GRPC,), jnp.int32),
            pltpu.VMEM((CHUNK, HALF), jnp.float32),
            pltpu.VMEM((CHUNK, HALF), jnp.float32),
            pltpu.VMEM((CHUNK, HALF), jnp.float32),
            pltpu.VMEM_SHARED((ACC_ROWS, HALF), jnp.float32),
            pltpu.SemaphoreType.DMA,
            pltpu.SemaphoreType.DMA,
            pltpu.SemaphoreType.DMA,
            pltpu.SemaphoreType.DMA,
            pltpu.SemaphoreType.DMA,
        ],
    )(_segsum_body)


def _segsum_body(src_hbm, dst_hbm, z_hbm, ya_hbm, yb_hbm, s0_hbm, s1_hbm,
                 srcr, dstr, rows0, rows1, rows2, acc, gsem0, gsem1, gsem2,
                 ssem, isem):
    c = lax.axis_index("c")
    s = lax.axis_index("s")
    base = s * EPT_PAD

    pltpu.sync_copy(z_hbm, acc.at[pl.ds(s * ZROWS, ZROWS)])
    pltpu.sync_copy(src_hbm.at[pl.ds(base, GRPC)], srcr.at[pl.ds(0, GRPC)])
    pltpu.sync_copy(dst_hbm.at[pl.ds(base, GRPC)], dstr.at[pl.ds(0, GRPC)])
    plsc.subcore_barrier()

    def run(y_hbm):
        bufs = (rows0, rows1, rows2)
        gsems = (gsem0, gsem1, gsem2)

        def roff(k, ch):
            return pl.multiple_of(lax.rem(k, 2) * GRPC + ch * CHUNK, 8)

        def g(k, ch, b):
            idx = srcr.at[pl.ds(roff(k, ch), CHUNK)]
            return pltpu.make_async_copy(y_hbm.at[idx], bufs[b], gsems[b])

        def didx(k, ch):
            return dstr.at[pl.ds(roff(k, ch), CHUNK)]

        def sc(k, ch, b):
            return pltpu.make_async_copy(bufs[b], acc.at[didx(k, ch)], ssem)

        def refill(gn, start):
            hb = pl.multiple_of(base + gn * GRPC, 8)
            ho = pl.multiple_of(lax.rem(gn, 2) * GRPC, 8)
            for hbm, ring in ((src_hbm, srcr), (dst_hbm, dstr)):
                d = pltpu.make_async_copy(hbm.at[pl.ds(hb, GRPC)],
                                          ring.at[pl.ds(ho, GRPC)], isem)
                d.start() if start else d.wait()

        g(0, 0, 0).start()
        g(0, 1, 1).start()

        def body(k, _):
            g(k, 0, 0).wait()

            @pl.when(k >= 1)
            def _():
                sc(k - 1, 2, 2).wait()

            @pl.when(k + 1 < NGROUP)
            def _():
                refill(k + 1, True)

            g(k, 2, 2).start()
            pltpu.async_copy(bufs[0], acc.at[didx(k, 0)], ssem, add=True)
            g(k, 1, 1).wait()
            sc(k, 0, 0).wait()

            @pl.when(k + 1 < NGROUP)
            def _():
                refill(k + 1, False)
                g(k + 1, 0, 0).start()

            pltpu.async_copy(bufs[1], acc.at[didx(k, 1)], ssem, add=True)
            g(k, 2, 2).wait()
            sc(k, 1, 1).wait()

            @pl.when(k + 1 < NGROUP)
            def _():
                g(k + 1, 1, 1).start()

            pltpu.async_copy(bufs[2], acc.at[didx(k, 2)], ssem, add=True)
            return ()

        lax.fori_loop(0, NGROUP, body, (), unroll=False)
        sc(NGROUP - 1, 2, 2).wait()

    @pl.when(c == 0)
    def _():
        run(ya_hbm)

    @pl.when(c == 1)
    def _():
        run(yb_hbm)

    plsc.subcore_barrier()

    @pl.when(c == 0)
    def _():
        _copy_out(acc, s0_hbm, s)

    @pl.when(c == 1)
    def _():
        _copy_out(acc, s1_hbm, s)


DEG_RING = 4


@functools.cache
def _deg_kernel():
    mesh = plsc.VectorSubcoreMesh(core_axis_name="c", subcore_axis_name="s",
                                  num_cores=NC, num_subcores=NS)
    return functools.partial(
        pl.kernel,
        out_type=(jax.ShapeDtypeStruct((N, HALF), jnp.float32),
                  jax.ShapeDtypeStruct((N, HALF), jnp.float32)),
        mesh=mesh,
        scratch_types=[
            pltpu.VMEM((NCHUNK, CHUNK), jnp.int32),
            pltpu.VMEM((CHUNK, HALF), jnp.float32),
            pltpu.VMEM_SHARED((ACC_ROWS, HALF), jnp.float32),
            pltpu.SemaphoreType.DMA,
        ],
    )(_deg_body)


def _deg_body(dst_hbm, ones_hbm, z_hbm, da_hbm, db_hbm, dstv, onesv, acc, ssem):
    c = lax.axis_index("c")
    s = lax.axis_index("s")

    pltpu.sync_copy(z_hbm, acc.at[pl.ds(s * ZROWS, ZROWS)])
    pltpu.sync_copy(ones_hbm, onesv)
    pltpu.sync_copy(dst_hbm.at[s], dstv)
    plsc.subcore_barrier()

    lo = c * (NCHUNK // 2)
    hi = lax.select(c == 0, NCHUNK // 2, NCHUNK)

    def scatter(j):
        return pltpu.make_async_copy(onesv, acc.at[dstv.at[j]], ssem)

    def body(j, _):
        @pl.when(j - DEG_RING >= lo)
        def _():
            scatter(j - DEG_RING).wait()

        pltpu.async_copy(onesv, acc.at[dstv.at[j]], ssem, add=True)
        return ()

    lax.fori_loop(lo, hi, body, (), unroll=False)

    def drain(j, _):
        @pl.when(j >= lo)
        def _():
            scatter(j).wait()
        return ()

    lax.fori_loop(hi - DEG_RING, hi, drain, (), unroll=False)
    plsc.subcore_barrier()

    @pl.when(c == 0)
    def _():
        _copy_out(acc, da_hbm, s)

    @pl.when(c == 1)
    def _():
        _copy_out(acc, db_hbm, s)


def _silu(v):
    return v * jax.nn.sigmoid(v)


def _bdot(a, w):
    return jnp.dot(a.astype(jnp.bfloat16), w.astype(jnp.bfloat16),
                   preferred_element_type=jnp.float32)


def _dinv_of(da_blk, db_blk):
    return lax.rsqrt(da_blk[:, 0:1] + db_blk[:, 0:1] + 1.0)


def _tc_in_kernel(x_r, da_r, db_r, wi_r, bi_r, wg_r, h_r, ya_r, yb_r):
    dinv = _dinv_of(da_r, db_r)
    h = _bdot(x_r[...], wi_r[...]) + bi_r[...]
    xw = _bdot(h, wg_r[...])
    y = xw * dinv
    h_r[...] = h
    ya_r[...] = y[:, :HALF]
    yb_r[...] = y[:, HALF:]


def _post_common(h_r, s0_r, s1_r, ya_r, yb_r, da_r, db_r, bg_r, bng_r, bnb_r,
                 lng_r, lnb_r, w1_r, w2_r):
    dinv = _dinv_of(da_r, db_r)
    S = jnp.concatenate([s0_r[...], s1_r[...]], axis=1)
    y = jnp.concatenate([ya_r[...], yb_r[...]], axis=1)
    agg = dinv * (S + y) + bg_r[...]
    bn = agg * lax.rsqrt(jnp.float32(1.0 + 1e-5)) * bng_r[...] + bnb_r[...]
    hm = _silu(bn) + h_r[...]
    mu = jnp.mean(hm, axis=-1, keepdims=True)
    var = jnp.mean((hm - mu) ** 2, axis=-1, keepdims=True)
    xn = (hm - mu) * lax.rsqrt(var + 1e-5) * lng_r[...] + lnb_r[...]
    u = _silu(_bdot(xn, w1_r[...]))
    return _bdot(u, w2_r[...]) + hm


def _tc_mid_kernel(h_r, s0_r, s1_r, ya_r, yb_r, da_r, db_r, bg_r, bng_r, bnb_r,
                   lng_r, lnb_r, w1_r, w2_r, wgn_r,
                   ho_r, yao_r, ybo_r):
    h2 = _post_common(h_r, s0_r, s1_r, ya_r, yb_r, da_r, db_r, bg_r, bng_r,
                      bnb_r, lng_r, lnb_r, w1_r, w2_r)
    dinv = _dinv_of(da_r, db_r)
    y2 = _bdot(h2, wgn_r[...]) * dinv
    ho_r[...] = h2
    yao_r[...] = y2[:, :HALF]
    ybo_r[...] = y2[:, HALF:]


def _tc_last_kernel(h_r, s0_r, s1_r, ya_r, yb_r, da_r, db_r, bg_r, bng_r,
                    bnb_r, lng_r, lnb_r, w1_r, w2_r, wo_r, bo_r, out_r):
    h2 = _post_common(h_r, s0_r, s1_r, ya_r, yb_r, da_r, db_r, bg_r, bng_r,
                      bnb_r, lng_r, lnb_r, w1_r, w2_r)
    out_r[...] = _bdot(h2, wo_r[...]) + bo_r[...]


def _row_spec(w):
    return pl.BlockSpec((BT, w), lambda i: (i, 0))


def _full_spec(r, w):
    return pl.BlockSpec((r, w), lambda i: (0, 0))


def _sds(r, w):
    return jax.ShapeDtypeStruct((r, w), jnp.float32)


def kernel(x, edge_index, W_in, b_in, W_gcn, b_gcn, bn_gamma, bn_beta,
           ln_gamma, ln_beta, W1, W2, W_out, b_out):
    src = edge_index[0].reshape(NS, EPT)
    dst = edge_index[1].reshape(NS, EPT)
    src_pad = jnp.pad(src, ((0, 0), (0, EPT_PAD - EPT))).reshape(-1)
    dst_pad = jnp.pad(dst, ((0, 0), (0, EPT_PAD - EPT)), constant_values=N)
    dst_pad3 = dst_pad.reshape(NS, NCHUNK, CHUNK)
    dst_pad = dst_pad.reshape(-1)
    z_half = jnp.zeros((ZROWS, HALF), jnp.float32)
    ones_chunk = jnp.ones((CHUNK, HALF), jnp.float32)

    da, db = _deg_kernel()(dst_pad3, ones_chunk, z_half)

    b_in2 = b_in.reshape(1, D)
    b_out2 = b_out.reshape(1, D)

    h, ya, yb = pl.pallas_call(
        _tc_in_kernel,
        grid=(GRID,),
        in_specs=[_row_spec(D), _row_spec(HALF), _row_spec(HALF),
                  _full_spec(D, D), _full_spec(1, D), _full_spec(D, D)],
        out_specs=[_row_spec(D), _row_spec(HALF), _row_spec(HALF)],
        out_shape=[_sds(N, D), _sds(N, HALF), _sds(N, HALF)],
    )(x, da, db, W_in, b_in2, W_gcn[0])

    mid = pl.pallas_call(
        _tc_mid_kernel,
        grid=(GRID,),
        in_specs=[_row_spec(D)] + [_row_spec(HALF)] * 6
                 + [_full_spec(1, D)] * 5
                 + [_full_spec(D, D)] * 3,
        out_specs=[_row_spec(D), _row_spec(HALF), _row_spec(HALF)],
        out_shape=[_sds(N, D), _sds(N, HALF), _sds(N, HALF)],
    )

    last = pl.pallas_call(
        _tc_last_kernel,
        grid=(GRID,),
        in_specs=[_row_spec(D)] + [_row_spec(HALF)] * 6
                 + [_full_spec(1, D)] * 5
                 + [_full_spec(D, D)] * 2
                 + [_full_spec(D, D), _full_spec(1, D)],
        out_specs=_row_spec(D),
        out_shape=_sds(N, D),
    )

    for i in range(L):
        s0, s1 = _segsum_kernel()(src_pad, dst_pad, z_half, ya, yb)
        norms = (b_gcn[i].reshape(1, D), bn_gamma[i].reshape(1, D),
                 bn_beta[i].reshape(1, D), ln_gamma[i].reshape(1, D),
                 ln_beta[i].reshape(1, D))
        if i < L - 1:
            h, ya, yb = mid(h, s0, s1, ya, yb, da, db, *norms,
                            W1[i], W2[i], W_gcn[i + 1])
        else:
            out = last(h, s0, s1, ya, yb, da, db, *norms,
                       W1[i], W2[i], W_out, b_out2)
    return out
